# Initial kernel scaffold; baseline (speedup 1.0000x reference)
import jax
import jax.numpy as jnp
from jax.experimental import pallas as pl


def kernel(shape_id, colour_id, pos_id, edge_index, batch, shape_table, col_table, pos_table, W1l, b1l, W1r, g1, be1, W2l, b2l, W2r, g2, be2, Wout, bout):
    raise NotImplementedError("write your pallas kernel here")


# speedup vs baseline: 3.8529x; 3.8529x over previous
import functools

import jax
import jax.numpy as jnp
from jax import lax
from jax.experimental import pallas as pl
from jax.experimental.pallas import tpu as pltpu
from jax.experimental.pallas import tpu_sc as plsc

N = 50000
E = 800000
HID = 64
HH = 32
N_SHAPE = 64
N_COL = 32
POS_V = 4097
N_GRAPHS = 128
N_CLASSES = 2

NC = 2
NS = 16
NW = NC * NS
CH = 128

B_NODE = 3328
N_PAD = NS * B_NODE
PAD_NODE = N_PAD - 1

EPT = 50048
E_PAD = NS * EPT

DRAIN_ROWS = N_PAD // NS
DRAIN_CHUNKS = DRAIN_ROWS // CH

BM = 2000
GRID = N // BM

_mesh = plsc.VectorSubcoreMesh(
    core_axis_name="c", subcore_axis_name="s", num_cores=NC, num_subcores=NS)
_sc_params = pltpu.CompilerParams(use_tc_tiling_on_sc=False)


def _worker():
    cidx = lax.axis_index("c")
    sidx = lax.axis_index("s")
    return cidx, sidx


def _embed_body(sid, cid, pid, shp_t, col_t, pos_t, x_out,
                si, ci, pi, rows, sem):
    cidx, sidx = _worker()
    base = sidx * B_NODE
    pltpu.sync_copy(sid.at[cidx, pl.ds(base, B_NODE)], si)
    pltpu.sync_copy(cid.at[cidx, pl.ds(base, B_NODE)], ci)
    pltpu.sync_copy(pid.at[cidx, pl.ds(base, B_NODE)], pi)

    def chunk(j, _):
        off = j * CH
        pltpu.sync_copy(shp_t.at[si.at[pl.ds(off, CH)]], rows)
        pltpu.sync_copy(col_t.at[ci.at[pl.ds(off, CH)]], rows, add=True)
        pltpu.sync_copy(pos_t.at[pi.at[pl.ds(off, CH)]], rows, add=True)
        pltpu.sync_copy(rows, x_out.at[cidx, pl.ds(base + off, CH)])
        return 0

    lax.fori_loop(0, B_NODE // CH, chunk, 0)
    del sem


_embed = pl.kernel(
    _embed_body,
    out_type=jax.ShapeDtypeStruct((NC, N_PAD, HH), jnp.float32),
    mesh=_mesh,
    scratch_types=[
        pltpu.VMEM((B_NODE,), jnp.int32),
        pltpu.VMEM((B_NODE,), jnp.int32),
        pltpu.VMEM((B_NODE,), jnp.int32),
        pltpu.VMEM((CH, HH), jnp.float32),
        pltpu.SemaphoreType.DMA,
    ],
    compiler_params=_sc_params,
)


def _edge_body(emit_deg, xs, src2, dst, *args):
    if emit_deg:
        (agg_out, deg_out, acc, dacc, sbuf, dbuf, mbuf, zbuf, ones, zb1) = args
    else:
        (agg_out, acc, sbuf, dbuf, mbuf, zbuf, ones, zb1) = args
        dacc = deg_out = None
    cidx, sidx = _worker()

    def zrow(r, _):
        zbuf[r, pl.ds(0, 16)] = jnp.zeros((16,), jnp.float32)
        zbuf[r, pl.ds(16, 16)] = jnp.zeros((16,), jnp.float32)
        return 0
    lax.fori_loop(0, CH, zrow, 0)

    def onesrow(k, _):
        ones[pl.ds(k * 16, 16)] = jnp.ones((16,), jnp.float32)
        zb1[pl.ds(k * 16, 16)] = jnp.zeros((16,), jnp.float32)
        return 0
    lax.fori_loop(0, CH // 16, onesrow, 0)

    def zchunk(k, _):
        row = sidx * DRAIN_ROWS + k * CH
        pltpu.sync_copy(zbuf, acc.at[pl.ds(row, CH)])
        if emit_deg:
            pltpu.sync_copy(zb1, dacc.at[pl.ds(row, CH)])
        return 0
    lax.fori_loop(0, DRAIN_CHUNKS, zchunk, 0)
    plsc.subcore_barrier()

    def echunk(j, _):
        ebase = sidx * EPT + j * CH
        pltpu.sync_copy(src2.at[cidx, pl.ds(ebase, CH)], sbuf)
        pltpu.sync_copy(dst.at[pl.ds(ebase, CH)], dbuf)
        pltpu.sync_copy(xs.at[sbuf], mbuf)
        pltpu.sync_copy(mbuf, acc.at[dbuf], add=True)
        if emit_deg:
            @pl.when(cidx == 0)
            def _():
                pltpu.sync_copy(ones, dacc.at[dbuf], add=True)
        return 0
    lax.fori_loop(0, EPT // CH, echunk, 0)
    plsc.subcore_barrier()

    def drain(k, _):
        row = sidx * DRAIN_ROWS + k * CH
        pltpu.sync_copy(acc.at[pl.ds(row, CH)], mbuf)
        pltpu.sync_copy(mbuf, agg_out.at[cidx, pl.ds(row, CH)])
        if emit_deg:
            @pl.when(cidx == 0)
            def _():
                pltpu.sync_copy(dacc.at[pl.ds(row, CH)],
                                deg_out.at[pl.ds(row, CH)])
        return 0
    lax.fori_loop(0, DRAIN_CHUNKS, drain, 0)


def _make_edge(emit_deg):
    outs = [jax.ShapeDtypeStruct((NC, N_PAD, HH), jnp.float32)]
    scratch = [pltpu.VMEM_SHARED((N_PAD, HH), jnp.float32)]
    if emit_deg:
        outs.append(jax.ShapeDtypeStruct((N_PAD,), jnp.float32))
        scratch.append(pltpu.VMEM_SHARED((N_PAD,), jnp.float32))
    scratch += [
        pltpu.VMEM((CH,), jnp.int32),
        pltpu.VMEM((CH,), jnp.int32),
        pltpu.VMEM((CH, HH), jnp.float32),
        pltpu.VMEM((CH, HH), jnp.float32),
        pltpu.VMEM((CH,), jnp.float32),
        pltpu.VMEM((CH,), jnp.float32),
    ]
    return pl.kernel(
        functools.partial(_edge_body, emit_deg),
        out_type=tuple(outs) if emit_deg else outs[0],
        mesh=_mesh,
        scratch_types=scratch,
        compiler_params=_sc_params,
    )


_edge_deg = _make_edge(True)
_edge = _make_edge(False)


def _dense_body(x_ref, a_ref, d_ref, wl_ref, bl_ref, wr_ref, y_ref, st_ref):
    i = pl.program_id(0)
    xb = jnp.concatenate([x_ref[0], x_ref[1]], axis=-1)
    ab = jnp.concatenate([a_ref[0], a_ref[1]], axis=-1)
    inv = 1.0 / jnp.maximum(d_ref[...], 1.0)
    ab = ab * inv
    y = (jnp.dot(ab, wl_ref[...], preferred_element_type=jnp.float32)
         + jnp.dot(xb, wr_ref[...], preferred_element_type=jnp.float32)
         + bl_ref[...])
    y_ref[...] = y

    @pl.when(i == 0)
    def _():
        st_ref[...] = jnp.zeros((8, HID), jnp.float32)
    s = jnp.sum(y, axis=0, keepdims=True)
    ss = jnp.sum(y * y, axis=0, keepdims=True)
    st_ref[...] += jnp.concatenate(
        [s, ss, jnp.zeros((6, HID), jnp.float32)], axis=0)


_dense = pl.pallas_call(
    _dense_body,
    grid=(GRID,),
    in_specs=[
        pl.BlockSpec((NC, BM, HH), lambda i: (0, i, 0)),
        pl.BlockSpec((NC, BM, HH), lambda i: (0, i, 0)),
        pl.BlockSpec((BM, 1), lambda i: (i, 0)),
        pl.BlockSpec((HID, HID), lambda i: (0, 0)),
        pl.BlockSpec((1, HID), lambda i: (0, 0)),
        pl.BlockSpec((HID, HID), lambda i: (0, 0)),
    ],
    out_specs=[
        pl.BlockSpec((BM, HID), lambda i: (i, 0)),
        pl.BlockSpec((8, HID), lambda i: (0, 0)),
    ],
    out_shape=[
        jax.ShapeDtypeStruct((N, HID), jnp.float32),
        jax.ShapeDtypeStruct((8, HID), jnp.float32),
    ],
)


def _bn(y, st_ref, g_ref, be_ref):
    st = st_ref[...]
    mu = st[0:1, :] * (1.0 / N)
    var = st[1:2, :] * (1.0 / N) - mu * mu
    inv = lax.rsqrt(var + 1e-5) * g_ref[...]
    return jnp.maximum((y - mu) * inv + be_ref[...], 0.0)


def _bnrelu_body(y_ref, st_ref, g_ref, be_ref, h_ref):
    h = _bn(y_ref[...], st_ref, g_ref, be_ref)
    h_ref[0] = h[:, :HH]
    h_ref[1] = h[:, HH:]


_bnrelu = pl.pallas_call(
    _bnrelu_body,
    grid=(GRID,),
    in_specs=[
        pl.BlockSpec((BM, HID), lambda i: (i, 0)),
        pl.BlockSpec((8, HID), lambda i: (0, 0)),
        pl.BlockSpec((1, HID), lambda i: (0, 0)),
        pl.BlockSpec((1, HID), lambda i: (0, 0)),
    ],
    out_specs=pl.BlockSpec((NC, BM, HH), lambda i: (0, i, 0)),
    out_shape=jax.ShapeDtypeStruct((NC, N_PAD, HH), jnp.float32),
)


def _pool_body(y_ref, st_ref, g_ref, be_ref, b_ref, wo_ref, bo_ref,
               out_ref, pool_ref):
    i = pl.program_id(0)
    h = _bn(y_ref[...], st_ref, g_ref, be_ref)

    @pl.when(i == 0)
    def _():
        pool_ref[...] = jnp.zeros((N_GRAPHS, HID), jnp.float32)

    gids = lax.broadcasted_iota(jnp.int32, (BM, N_GRAPHS), 1)
    oh = (b_ref[...] == gids).astype(jnp.float32)
    pool_ref[...] += lax.dot_general(
        oh, h, (((0,), (0,)), ((), ())), preferred_element_type=jnp.float32)

    @pl.when(i == GRID - 1)
    def _():
        out_ref[...] = (
            jnp.dot(pool_ref[...], wo_ref[...],
                    preferred_element_type=jnp.float32) + bo_ref[...])


_pool = pl.pallas_call(
    _pool_body,
    grid=(GRID,),
    in_specs=[
        pl.BlockSpec((BM, HID), lambda i: (i, 0)),
        pl.BlockSpec((8, HID), lambda i: (0, 0)),
        pl.BlockSpec((1, HID), lambda i: (0, 0)),
        pl.BlockSpec((1, HID), lambda i: (0, 0)),
        pl.BlockSpec((BM, 1), lambda i: (i, 0)),
        pl.BlockSpec((HID, N_CLASSES), lambda i: (0, 0)),
        pl.BlockSpec((1, N_CLASSES), lambda i: (0, 0)),
    ],
    out_specs=pl.BlockSpec((N_GRAPHS, N_CLASSES), lambda i: (0, 0)),
    out_shape=jax.ShapeDtypeStruct((N_GRAPHS, N_CLASSES), jnp.float32),
    scratch_shapes=[pltpu.VMEM((N_GRAPHS, HID), jnp.float32)],
)


def _split_table(t):
    return jnp.concatenate([t[:, :HH], t[:, HH:]], axis=0)


def _core_ids(ids, v):
    p = jnp.zeros((N_PAD - N,), jnp.int32)
    idp = jnp.concatenate([ids.astype(jnp.int32), p])
    return jnp.stack([idp, idp + v])


def kernel(shape_id, colour_id, pos_id, edge_index, batch, shape_table,
           col_table, pos_table, W1l, b1l, W1r, g1, be1, W2l, b2l, W2r,
           g2, be2, Wout, bout):
    sid2 = _core_ids(shape_id, N_SHAPE)
    cid2 = _core_ids(colour_id, N_COL)
    pid2 = _core_ids(pos_id, POS_V)
    shp_t = _split_table(shape_table)
    col_t = _split_table(col_table)
    pos_t = _split_table(pos_table)

    epad = jnp.full((2, E_PAD - E), PAD_NODE, jnp.int32)
    ei = jnp.concatenate([edge_index.astype(jnp.int32), epad], axis=1)
    src2 = jnp.stack([ei[0], ei[0] + N_PAD])
    dst = ei[1]

    b1l2 = b1l.reshape(1, HID)
    b2l2 = b2l.reshape(1, HID)
    g1_2, be1_2 = g1.reshape(1, HID), be1.reshape(1, HID)
    g2_2, be2_2 = g2.reshape(1, HID), be2.reshape(1, HID)
    bo2 = bout.reshape(1, N_CLASSES)
    batch2 = batch.astype(jnp.int32).reshape(N, 1)

    xs = _embed(sid2, cid2, pid2, shp_t, col_t, pos_t)
    agg1, deg = _edge_deg(xs.reshape(NC * N_PAD, HH), src2, dst)
    deg2 = deg.reshape(N_PAD, 1)
    y1, st1 = _dense(xs, agg1, deg2, W1l, b1l2, W1r)
    h1 = _bnrelu(y1, st1, g1_2, be1_2)
    agg2 = _edge(h1.reshape(NC * N_PAD, HH), src2, dst)
    y2, st2 = _dense(h1, agg2, deg2, W2l, b2l2, W2r)
    return _pool(y2, st2, g2_2, be2_2, batch2, Wout, bo2)


# speedup vs baseline: 6.7658x; 1.7560x over previous
import functools

import jax
import jax.numpy as jnp
from jax import lax
from jax.experimental import pallas as pl
from jax.experimental.pallas import tpu as pltpu
from jax.experimental.pallas import tpu_sc as plsc

N = 50000
E = 800000
HID = 64
HH = 32
N_SHAPE = 64
N_COL = 32
POS_V = 4097
N_GRAPHS = 128
N_CLASSES = 2

NC = 2
NS = 16
NW = NC * NS
CH = 128

B_NODE = 3328
N_PAD = NS * B_NODE
PAD_NODE = N_PAD - 1

ECH = 392
EPT = ECH * CH
E_PAD = NS * EPT

DRAIN_ROWS = N_PAD // NS
DRAIN_CHUNKS = DRAIN_ROWS // CH

BM = 2000
GRID = N // BM

_mesh = plsc.VectorSubcoreMesh(
    core_axis_name="c", subcore_axis_name="s", num_cores=NC, num_subcores=NS)
_sc_params = pltpu.CompilerParams(use_tc_tiling_on_sc=False)


def _worker():
    cidx = lax.axis_index("c")
    sidx = lax.axis_index("s")
    return cidx, sidx


def _embed_body(sid, cid, pid, shp_t, col_t, pos_t, x_out,
                si, ci, pi, rows, sem):
    cidx, sidx = _worker()
    base = sidx * B_NODE
    pltpu.sync_copy(sid.at[cidx, pl.ds(base, B_NODE)], si)
    pltpu.sync_copy(cid.at[cidx, pl.ds(base, B_NODE)], ci)
    pltpu.sync_copy(pid.at[cidx, pl.ds(base, B_NODE)], pi)

    def chunk(j, _):
        off = j * CH
        pltpu.sync_copy(shp_t.at[si.at[pl.ds(off, CH)]], rows)
        pltpu.sync_copy(col_t.at[ci.at[pl.ds(off, CH)]], rows, add=True)
        pltpu.sync_copy(pos_t.at[pi.at[pl.ds(off, CH)]], rows, add=True)
        pltpu.sync_copy(rows, x_out.at[cidx, pl.ds(base + off, CH)])
        return 0

    lax.fori_loop(0, B_NODE // CH, chunk, 0)
    del sem


_embed = pl.kernel(
    _embed_body,
    out_type=jax.ShapeDtypeStruct((NC, N_PAD, HH), jnp.float32),
    mesh=_mesh,
    scratch_types=[
        pltpu.VMEM((B_NODE,), jnp.int32),
        pltpu.VMEM((B_NODE,), jnp.int32),
        pltpu.VMEM((B_NODE,), jnp.int32),
        pltpu.VMEM((CH, HH), jnp.float32),
        pltpu.SemaphoreType.DMA,
    ],
    compiler_params=_sc_params,
)


def _edge_body(emit_deg, xs, src1, dst1, *args):
    if emit_deg:
        (agg_out, deg_out, acc, dacc, sb0, sb1, db0, db1,
         mb0, mb1, zbuf, ones, zb1, si0, si1, sg0, sg1) = args
    else:
        (agg_out, acc, sb0, sb1, db0, db1,
         mb0, mb1, zbuf, ones, zb1, si0, si1, sg0, sg1) = args
        dacc = deg_out = None
    sb = (sb0, sb1)
    db = (db0, db1)
    mbufs = (mb0, mb1)
    sem_i = (si0, si1)
    sem_g = (sg0, sg1)
    cidx, sidx = _worker()

    def zrow(r, _):
        zbuf[r, pl.ds(0, 16)] = jnp.zeros((16,), jnp.float32)
        zbuf[r, pl.ds(16, 16)] = jnp.zeros((16,), jnp.float32)
        return 0
    lax.fori_loop(0, CH, zrow, 0)

    def onesrow(k, _):
        ones[pl.ds(k * 16, 16)] = jnp.ones((16,), jnp.float32)
        zb1[pl.ds(k * 16, 16)] = jnp.zeros((16,), jnp.float32)
        return 0
    lax.fori_loop(0, CH // 16, onesrow, 0)

    def zchunk(k, _):
        row = sidx * DRAIN_ROWS + k * CH
        pltpu.sync_copy(zbuf, acc.at[pl.ds(row, CH)])
        if emit_deg:
            pltpu.sync_copy(zb1, dacc.at[pl.ds(row, CH)])
        return 0
    lax.fori_loop(0, DRAIN_CHUNKS, zchunk, 0)
    plsc.subcore_barrier()

    ebase = sidx * EPT
    soff = cidx * N_PAD

    def issue_idx(j, b):
        pltpu.async_copy(src1.at[pl.ds(ebase + j * CH, CH)], sb[b], sem_i[b])
        pltpu.async_copy(dst1.at[pl.ds(ebase + j * CH, CH)], db[b], sem_i[b])

    def wait_idx(b):
        pltpu.make_async_copy(src1.at[pl.ds(0, CH)], sb[b], sem_i[b]).wait()
        pltpu.make_async_copy(dst1.at[pl.ds(0, CH)], db[b], sem_i[b]).wait()

    def issue_gather(b):
        for k in range(CH // 16):
            sl = pl.ds(k * 16, 16)
            sb[b][sl] = sb[b][sl] + soff
        pltpu.async_copy(xs.at[sb[b]], mbufs[b], sem_g[b])

    def wait_gather(b):
        pltpu.make_async_copy(xs.at[sb[b]], mbufs[b], sem_g[b]).wait()

    issue_idx(0, 0)
    issue_idx(1, 1)
    wait_idx(0)
    issue_gather(0)

    def echunk(kk, _):
        for b in range(2):
            j = kk * 2 + b
            bn = 1 - b

            @pl.when(j + 1 < ECH)
            def _():
                wait_idx(bn)
                issue_gather(bn)
            wait_gather(b)
            pltpu.sync_copy(mbufs[b], acc.at[db[b]], add=True)
            if emit_deg:
                @pl.when(cidx == 0)
                def _():
                    pltpu.sync_copy(ones, dacc.at[db[b]], add=True)

            @pl.when(j + 2 < ECH)
            def _():
                issue_idx(j + 2, b)
        return 0
    lax.fori_loop(0, ECH // 2, echunk, 0)
    plsc.subcore_barrier()

    def drain(k, _):
        row = sidx * DRAIN_ROWS + k * CH
        pltpu.sync_copy(acc.at[pl.ds(row, CH)], mb0)
        pltpu.sync_copy(mb0, agg_out.at[cidx, pl.ds(row, CH)])
        if emit_deg:
            @pl.when(cidx == 0)
            def _():
                pltpu.sync_copy(dacc.at[pl.ds(row, CH)],
                                deg_out.at[pl.ds(row, CH)])
        return 0
    lax.fori_loop(0, DRAIN_CHUNKS, drain, 0)


def _make_edge(emit_deg):
    outs = [jax.ShapeDtypeStruct((NC, N_PAD, HH), jnp.float32)]
    scratch = [pltpu.VMEM_SHARED((N_PAD, HH), jnp.float32)]
    if emit_deg:
        outs.append(jax.ShapeDtypeStruct((N_PAD,), jnp.float32))
        scratch.append(pltpu.VMEM_SHARED((N_PAD,), jnp.float32))
    scratch += [
        pltpu.VMEM((CH,), jnp.int32),
        pltpu.VMEM((CH,), jnp.int32),
        pltpu.VMEM((CH,), jnp.int32),
        pltpu.VMEM((CH,), jnp.int32),
        pltpu.VMEM((CH, HH), jnp.float32),
        pltpu.VMEM((CH, HH), jnp.float32),
        pltpu.VMEM((CH, HH), jnp.float32),
        pltpu.VMEM((CH,), jnp.float32),
        pltpu.VMEM((CH,), jnp.float32),
        pltpu.SemaphoreType.DMA,
        pltpu.SemaphoreType.DMA,
        pltpu.SemaphoreType.DMA,
        pltpu.SemaphoreType.DMA,
    ]
    return pl.kernel(
        functools.partial(_edge_body, emit_deg),
        out_type=tuple(outs) if emit_deg else outs[0],
        mesh=_mesh,
        scratch_types=scratch,
        compiler_params=_sc_params,
    )


_edge_deg = _make_edge(True)
_edge = _make_edge(False)


def _dense_body(x_ref, a_ref, d_ref, wl_ref, bl_ref, wr_ref, y_ref, st_ref):
    i = pl.program_id(0)
    xb = jnp.concatenate([x_ref[0], x_ref[1]], axis=-1)
    ab = jnp.concatenate([a_ref[0], a_ref[1]], axis=-1)
    inv = 1.0 / jnp.maximum(d_ref[...], 1.0)
    ab = ab * inv
    y = (jnp.dot(ab, wl_ref[...], preferred_element_type=jnp.float32)
         + jnp.dot(xb, wr_ref[...], preferred_element_type=jnp.float32)
         + bl_ref[...])
    y_ref[...] = y

    @pl.when(i == 0)
    def _():
        st_ref[...] = jnp.zeros((8, HID), jnp.float32)
    s = jnp.sum(y, axis=0, keepdims=True)
    ss = jnp.sum(y * y, axis=0, keepdims=True)
    st_ref[...] += jnp.concatenate(
        [s, ss, jnp.zeros((6, HID), jnp.float32)], axis=0)


_dense = pl.pallas_call(
    _dense_body,
    grid=(GRID,),
    in_specs=[
        pl.BlockSpec((NC, BM, HH), lambda i: (0, i, 0)),
        pl.BlockSpec((NC, BM, HH), lambda i: (0, i, 0)),
        pl.BlockSpec((BM, 1), lambda i: (i, 0)),
        pl.BlockSpec((HID, HID), lambda i: (0, 0)),
        pl.BlockSpec((1, HID), lambda i: (0, 0)),
        pl.BlockSpec((HID, HID), lambda i: (0, 0)),
    ],
    out_specs=[
        pl.BlockSpec((BM, HID), lambda i: (i, 0)),
        pl.BlockSpec((8, HID), lambda i: (0, 0)),
    ],
    out_shape=[
        jax.ShapeDtypeStruct((N, HID), jnp.float32),
        jax.ShapeDtypeStruct((8, HID), jnp.float32),
    ],
)


def _bn(y, st_ref, g_ref, be_ref):
    st = st_ref[...]
    mu = st[0:1, :] * (1.0 / N)
    var = st[1:2, :] * (1.0 / N) - mu * mu
    inv = lax.rsqrt(var + 1e-5) * g_ref[...]
    return jnp.maximum((y - mu) * inv + be_ref[...], 0.0)


def _bnrelu_body(y_ref, st_ref, g_ref, be_ref, h_ref):
    h = _bn(y_ref[...], st_ref, g_ref, be_ref)
    h_ref[0] = h[:, :HH]
    h_ref[1] = h[:, HH:]


_bnrelu = pl.pallas_call(
    _bnrelu_body,
    grid=(GRID,),
    in_specs=[
        pl.BlockSpec((BM, HID), lambda i: (i, 0)),
        pl.BlockSpec((8, HID), lambda i: (0, 0)),
        pl.BlockSpec((1, HID), lambda i: (0, 0)),
        pl.BlockSpec((1, HID), lambda i: (0, 0)),
    ],
    out_specs=pl.BlockSpec((NC, BM, HH), lambda i: (0, i, 0)),
    out_shape=jax.ShapeDtypeStruct((NC, N_PAD, HH), jnp.float32),
)


def _pool_body(y_ref, st_ref, g_ref, be_ref, b_ref, wo_ref, bo_ref,
               out_ref, pool_ref):
    i = pl.program_id(0)
    h = _bn(y_ref[...], st_ref, g_ref, be_ref)

    @pl.when(i == 0)
    def _():
        pool_ref[...] = jnp.zeros((N_GRAPHS, HID), jnp.float32)

    gids = lax.broadcasted_iota(jnp.int32, (BM, N_GRAPHS), 1)
    oh = (b_ref[...] == gids).astype(jnp.float32)
    pool_ref[...] += lax.dot_general(
        oh, h, (((0,), (0,)), ((), ())), preferred_element_type=jnp.float32)

    @pl.when(i == GRID - 1)
    def _():
        out_ref[...] = (
            jnp.dot(pool_ref[...], wo_ref[...],
                    preferred_element_type=jnp.float32) + bo_ref[...])


_pool = pl.pallas_call(
    _pool_body,
    grid=(GRID,),
    in_specs=[
        pl.BlockSpec((BM, HID), lambda i: (i, 0)),
        pl.BlockSpec((8, HID), lambda i: (0, 0)),
        pl.BlockSpec((1, HID), lambda i: (0, 0)),
        pl.BlockSpec((1, HID), lambda i: (0, 0)),
        pl.BlockSpec((BM, 1), lambda i: (i, 0)),
        pl.BlockSpec((HID, N_CLASSES), lambda i: (0, 0)),
        pl.BlockSpec((1, N_CLASSES), lambda i: (0, 0)),
    ],
    out_specs=pl.BlockSpec((N_GRAPHS, N_CLASSES), lambda i: (0, 0)),
    out_shape=jax.ShapeDtypeStruct((N_GRAPHS, N_CLASSES), jnp.float32),
    scratch_shapes=[pltpu.VMEM((N_GRAPHS, HID), jnp.float32)],
)


def _split_table(t):
    return jnp.concatenate([t[:, :HH], t[:, HH:]], axis=0)


def _core_ids(ids, v):
    p = jnp.zeros((N_PAD - N,), jnp.int32)
    idp = jnp.concatenate([ids.astype(jnp.int32), p])
    return jnp.stack([idp, idp + v])


def kernel(shape_id, colour_id, pos_id, edge_index, batch, shape_table,
           col_table, pos_table, W1l, b1l, W1r, g1, be1, W2l, b2l, W2r,
           g2, be2, Wout, bout):
    sid2 = _core_ids(shape_id, N_SHAPE)
    cid2 = _core_ids(colour_id, N_COL)
    pid2 = _core_ids(pos_id, POS_V)
    shp_t = _split_table(shape_table)
    col_t = _split_table(col_table)
    pos_t = _split_table(pos_table)

    epad = jnp.full((2, E_PAD - E), PAD_NODE, jnp.int32)
    ei = jnp.concatenate([edge_index.astype(jnp.int32), epad], axis=1)
    src1, dst1 = ei[0], ei[1]

    b1l2 = b1l.reshape(1, HID)
    b2l2 = b2l.reshape(1, HID)
    g1_2, be1_2 = g1.reshape(1, HID), be1.reshape(1, HID)
    g2_2, be2_2 = g2.reshape(1, HID), be2.reshape(1, HID)
    bo2 = bout.reshape(1, N_CLASSES)
    batch2 = batch.astype(jnp.int32).reshape(N, 1)

    xs = _embed(sid2, cid2, pid2, shp_t, col_t, pos_t)
    agg1, deg = _edge_deg(xs.reshape(NC * N_PAD, HH), src1, dst1)
    deg2 = deg.reshape(N_PAD, 1)
    y1, st1 = _dense(xs, agg1, deg2, W1l, b1l2, W1r)
    h1 = _bnrelu(y1, st1, g1_2, be1_2)
    agg2 = _edge(h1.reshape(NC * N_PAD, HH), src1, dst1)
    y2, st2 = _dense(h1, agg2, deg2, W2l, b2l2, W2r)
    return _pool(y2, st2, g2_2, be2_2, batch2, Wout, bo2)


# speedup vs baseline: 8.5965x; 1.2706x over previous
import functools

import jax
import jax.numpy as jnp
from jax import lax
from jax.experimental import pallas as pl
from jax.experimental.pallas import tpu as pltpu
from jax.experimental.pallas import tpu_sc as plsc

N = 50000
E = 800000
HID = 64
HH = 32
N_SHAPE = 64
N_COL = 32
POS_V = 4097
N_GRAPHS = 128
N_CLASSES = 2

NC = 2
NS = 16
NW = NC * NS
CH = 128

B_NODE = 3328
N_PAD = NS * B_NODE
PAD_NODE = N_PAD - 1

ECH = 392
EPT = ECH * CH
E_PAD = NS * EPT

DRAIN_ROWS = N_PAD // NS
DRAIN_CHUNKS = DRAIN_ROWS // CH

BM = 2000
GRID = N // BM

_mesh = plsc.VectorSubcoreMesh(
    core_axis_name="c", subcore_axis_name="s", num_cores=NC, num_subcores=NS)
_sc_params = pltpu.CompilerParams(use_tc_tiling_on_sc=False)


def _worker():
    cidx = lax.axis_index("c")
    sidx = lax.axis_index("s")
    return cidx, sidx


def _embed_body(sid, cid, pid, shp_t, col_t, pos_t, x_out, *scr):
    si, ci, pi = scr[0:3]
    r0 = scr[3:5]
    r1 = scr[5:7]
    r2 = scr[7:9]
    stb = scr[9:11]
    sg = scr[11:13]
    ss = scr[13:15]
    cidx, sidx = _worker()
    base = sidx * B_NODE
    pltpu.sync_copy(sid.at[cidx, pl.ds(base, B_NODE)], si)
    pltpu.sync_copy(cid.at[cidx, pl.ds(base, B_NODE)], ci)
    pltpu.sync_copy(pid.at[cidx, pl.ds(base, B_NODE)], pi)

    NCHK = B_NODE // CH

    def issue3(j, b):
        off = pl.ds(j * CH, CH)
        pltpu.async_copy(shp_t.at[si.at[off]], r0[b], sg[b])
        pltpu.async_copy(col_t.at[ci.at[off]], r1[b], sg[b])
        pltpu.async_copy(pos_t.at[pi.at[off]], r2[b], sg[b])

    def wait3(b):
        off = pl.ds(0, CH)
        pltpu.make_async_copy(shp_t.at[si.at[off]], r0[b], sg[b]).wait()
        pltpu.make_async_copy(col_t.at[ci.at[off]], r1[b], sg[b]).wait()
        pltpu.make_async_copy(pos_t.at[pi.at[off]], r2[b], sg[b]).wait()

    def wait_store(b):
        pltpu.make_async_copy(
            stb[b], x_out.at[cidx, pl.ds(base, CH)], ss[b]).wait()

    issue3(0, 0)
    issue3(1, 1)

    def chunk(kk, _):
        for b in range(2):
            j = kk * 2 + b
            wait3(b)

            @pl.when(j >= 2)
            def _():
                wait_store(b)

            def vadd(r, _):
                for h in range(2):
                    sl = pl.ds(h * 16, 16)
                    stb[b][r, sl] = (r0[b][r, sl] + r1[b][r, sl]
                                     + r2[b][r, sl])
                return 0
            lax.fori_loop(0, CH, vadd, 0)
            pltpu.async_copy(
                stb[b], x_out.at[cidx, pl.ds(base + j * CH, CH)], ss[b])

            @pl.when(j + 2 < NCHK)
            def _():
                issue3(j + 2, b)
        return 0

    lax.fori_loop(0, NCHK // 2, chunk, 0)
    wait_store(0)
    wait_store(1)


_embed = pl.kernel(
    _embed_body,
    out_type=jax.ShapeDtypeStruct((NC, N_PAD, HH), jnp.float32),
    mesh=_mesh,
    scratch_types=(
        [pltpu.VMEM((B_NODE,), jnp.int32)] * 3
        + [pltpu.VMEM((CH, HH), jnp.float32)] * 8
        + [pltpu.SemaphoreType.DMA] * 4
    ),
    compiler_params=_sc_params,
)


NB = 4


def _edge_body(xs, src1, dst1, agg_out, *rest):
    sb = rest[0:NB]
    db = rest[NB:2 * NB]
    mbufs = rest[2 * NB:3 * NB]
    acc, zbuf = rest[3 * NB:3 * NB + 2]
    sem_i = rest[3 * NB + 2:4 * NB + 2]
    sem_g = rest[4 * NB + 2:5 * NB + 2]
    cidx, sidx = _worker()

    def zrow(r, _):
        zbuf[r, pl.ds(0, 16)] = jnp.zeros((16,), jnp.float32)
        zbuf[r, pl.ds(16, 16)] = jnp.zeros((16,), jnp.float32)
        return 0
    lax.fori_loop(0, CH, zrow, 0)

    def zchunk(k, _):
        row = sidx * DRAIN_ROWS + k * CH
        pltpu.sync_copy(zbuf, acc.at[pl.ds(row, CH)])
        return 0
    lax.fori_loop(0, DRAIN_CHUNKS, zchunk, 0)
    plsc.subcore_barrier()

    ebase = sidx * EPT
    soff = cidx * N_PAD

    def issue_idx(j, b):
        pltpu.async_copy(src1.at[pl.ds(ebase + j * CH, CH)], sb[b], sem_i[b])
        pltpu.async_copy(dst1.at[pl.ds(ebase + j * CH, CH)], db[b], sem_i[b])

    def wait_idx(b):
        pltpu.make_async_copy(src1.at[pl.ds(0, CH)], sb[b], sem_i[b]).wait()
        pltpu.make_async_copy(dst1.at[pl.ds(0, CH)], db[b], sem_i[b]).wait()

    def issue_gather(b):
        for k in range(CH // 16):
            sl = pl.ds(k * 16, 16)
            sb[b][sl] = sb[b][sl] + soff
        pltpu.async_copy(xs.at[sb[b]], mbufs[b], sem_g[b])

    def wait_gather(b):
        pltpu.make_async_copy(xs.at[sb[b]], mbufs[b], sem_g[b]).wait()

    for b in range(NB):
        issue_idx(b, b)
    wait_idx(0)
    issue_gather(0)
    wait_idx(1)
    issue_gather(1)

    def echunk(kk, _):
        for b in range(NB):
            j = kk * NB + b

            @pl.when(j + 2 < ECH)
            def _():
                wait_idx((b + 2) % NB)
                issue_gather((b + 2) % NB)
            wait_gather(b)
            pltpu.sync_copy(mbufs[b], acc.at[db[b]], add=True)

            @pl.when(j + NB < ECH)
            def _():
                issue_idx(j + NB, b)
        return 0
    lax.fori_loop(0, ECH // NB, echunk, 0)
    plsc.subcore_barrier()

    def drain(k, _):
        row = sidx * DRAIN_ROWS + k * CH
        pltpu.sync_copy(acc.at[pl.ds(row, CH)],
                        agg_out.at[cidx, pl.ds(row, CH)])
        return 0
    lax.fori_loop(0, DRAIN_CHUNKS, drain, 0)


_edge = pl.kernel(
    _edge_body,
    out_type=jax.ShapeDtypeStruct((NC, N_PAD, HH), jnp.float32),
    mesh=_mesh,
    scratch_types=(
        [pltpu.VMEM((CH,), jnp.int32)] * (2 * NB)
        + [pltpu.VMEM((CH, HH), jnp.float32)] * NB
        + [pltpu.VMEM_SHARED((N_PAD, HH), jnp.float32)]
        + [pltpu.VMEM((CH, HH), jnp.float32)]
        + [pltpu.SemaphoreType.DMA] * (2 * NB)
    ),
    compiler_params=_sc_params,
)


DCH = E_PAD // 2 // NS // CH


def _deg_body(dst1, deg_out, db0, db1, dacc, ones, zb1, si0, si1):
    db = (db0, db1)
    sem_i = (si0, si1)
    cidx, sidx = _worker()

    def onesrow(k, _):
        ones[pl.ds(k * 16, 16)] = jnp.ones((16,), jnp.float32)
        zb1[pl.ds(k * 16, 16)] = jnp.zeros((16,), jnp.float32)
        return 0
    lax.fori_loop(0, CH // 16, onesrow, 0)

    def zchunk(k, _):
        row = sidx * DRAIN_ROWS + k * CH
        pltpu.sync_copy(zb1, dacc.at[pl.ds(row, CH)])
        return 0
    lax.fori_loop(0, DRAIN_CHUNKS, zchunk, 0)
    plsc.subcore_barrier()

    ebase = cidx * (E_PAD // 2) + sidx * DCH * CH

    def issue_idx(j, b):
        pltpu.async_copy(dst1.at[pl.ds(ebase + j * CH, CH)], db[b], sem_i[b])

    def wait_idx(b):
        pltpu.make_async_copy(dst1.at[pl.ds(0, CH)], db[b], sem_i[b]).wait()

    issue_idx(0, 0)
    issue_idx(1, 1)

    def dchunk(kk, _):
        for b in range(2):
            j = kk * 2 + b
            wait_idx(b)
            pltpu.sync_copy(ones, dacc.at[db[b]], add=True)

            @pl.when(j + 2 < DCH)
            def _():
                issue_idx(j + 2, b)
        return 0
    lax.fori_loop(0, DCH // 2, dchunk, 0)
    plsc.subcore_barrier()

    def drain(k, _):
        row = sidx * DRAIN_ROWS + k * CH
        pltpu.sync_copy(dacc.at[pl.ds(row, CH)],
                        deg_out.at[cidx, pl.ds(row, CH)])
        return 0
    lax.fori_loop(0, DRAIN_CHUNKS, drain, 0)


_deg = pl.kernel(
    _deg_body,
    out_type=jax.ShapeDtypeStruct((NC, N_PAD), jnp.float32),
    mesh=_mesh,
    scratch_types=(
        [pltpu.VMEM((CH,), jnp.int32)] * 2
        + [pltpu.VMEM_SHARED((N_PAD,), jnp.float32)]
        + [pltpu.VMEM((CH,), jnp.float32)] * 2
        + [pltpu.SemaphoreType.DMA] * 2
    ),
    compiler_params=_sc_params,
)


def _dense_body(x_ref, a_ref, d_ref, wl_ref, bl_ref, wr_ref, y_ref, st_ref):
    i = pl.program_id(0)
    xb = jnp.concatenate([x_ref[0], x_ref[1]], axis=-1)
    ab = jnp.concatenate([a_ref[0], a_ref[1]], axis=-1)
    inv = 1.0 / jnp.maximum(d_ref[0] + d_ref[1], 1.0)
    ab = ab * inv
    y = (jnp.dot(ab, wl_ref[...], preferred_element_type=jnp.float32)
         + jnp.dot(xb, wr_ref[...], preferred_element_type=jnp.float32)
         + bl_ref[...])
    y_ref[...] = y

    @pl.when(i == 0)
    def _():
        st_ref[...] = jnp.zeros((8, HID), jnp.float32)
    s = jnp.sum(y, axis=0, keepdims=True)
    ss = jnp.sum(y * y, axis=0, keepdims=True)
    st_ref[...] += jnp.concatenate(
        [s, ss, jnp.zeros((6, HID), jnp.float32)], axis=0)


_dense = pl.pallas_call(
    _dense_body,
    grid=(GRID,),
    in_specs=[
        pl.BlockSpec((NC, BM, HH), lambda i: (0, i, 0)),
        pl.BlockSpec((NC, BM, HH), lambda i: (0, i, 0)),
        pl.BlockSpec((NC, BM, 1), lambda i: (0, i, 0)),
        pl.BlockSpec((HID, HID), lambda i: (0, 0)),
        pl.BlockSpec((1, HID), lambda i: (0, 0)),
        pl.BlockSpec((HID, HID), lambda i: (0, 0)),
    ],
    out_specs=[
        pl.BlockSpec((BM, HID), lambda i: (i, 0)),
        pl.BlockSpec((8, HID), lambda i: (0, 0)),
    ],
    out_shape=[
        jax.ShapeDtypeStruct((N, HID), jnp.float32),
        jax.ShapeDtypeStruct((8, HID), jnp.float32),
    ],
)


def _bn(y, st_ref, g_ref, be_ref):
    st = st_ref[...]
    mu = st[0:1, :] * (1.0 / N)
    var = st[1:2, :] * (1.0 / N) - mu * mu
    inv = lax.rsqrt(var + 1e-5) * g_ref[...]
    return jnp.maximum((y - mu) * inv + be_ref[...], 0.0)


def _bnrelu_body(y_ref, st_ref, g_ref, be_ref, h_ref):
    h = _bn(y_ref[...], st_ref, g_ref, be_ref)
    h_ref[0] = h[:, :HH]
    h_ref[1] = h[:, HH:]


_bnrelu = pl.pallas_call(
    _bnrelu_body,
    grid=(GRID,),
    in_specs=[
        pl.BlockSpec((BM, HID), lambda i: (i, 0)),
        pl.BlockSpec((8, HID), lambda i: (0, 0)),
        pl.BlockSpec((1, HID), lambda i: (0, 0)),
        pl.BlockSpec((1, HID), lambda i: (0, 0)),
    ],
    out_specs=pl.BlockSpec((NC, BM, HH), lambda i: (0, i, 0)),
    out_shape=jax.ShapeDtypeStruct((NC, N_PAD, HH), jnp.float32),
)


def _pool_body(y_ref, st_ref, g_ref, be_ref, b_ref, wo_ref, bo_ref,
               out_ref, pool_ref):
    i = pl.program_id(0)
    h = _bn(y_ref[...], st_ref, g_ref, be_ref)

    @pl.when(i == 0)
    def _():
        pool_ref[...] = jnp.zeros((N_GRAPHS, HID), jnp.float32)

    gids = lax.broadcasted_iota(jnp.int32, (BM, N_GRAPHS), 1)
    oh = (b_ref[...] == gids).astype(jnp.float32)
    pool_ref[...] += lax.dot_general(
        oh, h, (((0,), (0,)), ((), ())), preferred_element_type=jnp.float32)

    @pl.when(i == GRID - 1)
    def _():
        out_ref[...] = (
            jnp.dot(pool_ref[...], wo_ref[...],
                    preferred_element_type=jnp.float32) + bo_ref[...])


_pool = pl.pallas_call(
    _pool_body,
    grid=(GRID,),
    in_specs=[
        pl.BlockSpec((BM, HID), lambda i: (i, 0)),
        pl.BlockSpec((8, HID), lambda i: (0, 0)),
        pl.BlockSpec((1, HID), lambda i: (0, 0)),
        pl.BlockSpec((1, HID), lambda i: (0, 0)),
        pl.BlockSpec((BM, 1), lambda i: (i, 0)),
        pl.BlockSpec((HID, N_CLASSES), lambda i: (0, 0)),
        pl.BlockSpec((1, N_CLASSES), lambda i: (0, 0)),
    ],
    out_specs=pl.BlockSpec((N_GRAPHS, N_CLASSES), lambda i: (0, 0)),
    out_shape=jax.ShapeDtypeStruct((N_GRAPHS, N_CLASSES), jnp.float32),
    scratch_shapes=[pltpu.VMEM((N_GRAPHS, HID), jnp.float32)],
)


def _split_table(t):
    return jnp.concatenate([t[:, :HH], t[:, HH:]], axis=0)


def _core_ids(ids, v):
    p = jnp.zeros((N_PAD - N,), jnp.int32)
    idp = jnp.concatenate([ids.astype(jnp.int32), p])
    return jnp.stack([idp, idp + v])


def kernel(shape_id, colour_id, pos_id, edge_index, batch, shape_table,
           col_table, pos_table, W1l, b1l, W1r, g1, be1, W2l, b2l, W2r,
           g2, be2, Wout, bout):
    sid2 = _core_ids(shape_id, N_SHAPE)
    cid2 = _core_ids(colour_id, N_COL)
    pid2 = _core_ids(pos_id, POS_V)
    shp_t = _split_table(shape_table)
    col_t = _split_table(col_table)
    pos_t = _split_table(pos_table)

    epad = jnp.full((2, E_PAD - E), PAD_NODE, jnp.int32)
    ei = jnp.concatenate([edge_index.astype(jnp.int32), epad], axis=1)
    src1, dst1 = ei[0], ei[1]

    b1l2 = b1l.reshape(1, HID)
    b2l2 = b2l.reshape(1, HID)
    g1_2, be1_2 = g1.reshape(1, HID), be1.reshape(1, HID)
    g2_2, be2_2 = g2.reshape(1, HID), be2.reshape(1, HID)
    bo2 = bout.reshape(1, N_CLASSES)
    batch2 = batch.astype(jnp.int32).reshape(N, 1)

    xs = _embed(sid2, cid2, pid2, shp_t, col_t, pos_t)
    agg1 = _edge(xs.reshape(NC * N_PAD, HH), src1, dst1)
    deg = _deg(dst1)
    deg2 = deg.reshape(NC, N_PAD, 1)
    y1, st1 = _dense(xs, agg1, deg2, W1l, b1l2, W1r)
    h1 = _bnrelu(y1, st1, g1_2, be1_2)
    agg2 = _edge(h1.reshape(NC * N_PAD, HH), src1, dst1)
    y2, st2 = _dense(h1, agg2, deg2, W2l, b2l2, W2r)
    return _pool(y2, st2, g2_2, be2_2, batch2, Wout, bo2)
